# baseline (speedup 1.0000x reference)
import functools

import jax
import jax.numpy as jnp
from jax import lax
from jax.experimental import pallas as pl
from jax.experimental.pallas import tpu as pltpu
from jax.experimental.pallas import tpu_sc as plsc

N = 10000
E = 320000
D = 128
H = 128
C = 10
G = 64

NC = 2
NS = 16
NW = NC * NS
EPW = E // NW
CH = 125
NCHUNK = EPW // CH
SCH = 8
NPAD = 10240
RT = NPAD // NS

@functools.lru_cache(maxsize=1)
def _vector_mesh():
    return plsc.VectorSubcoreMesh(
        core_axis_name="c", subcore_axis_name="s",
        num_cores=NC, num_subcores=NS)


@jax.jit
def _sc_degree(dst3):
    @functools.partial(
        pl.kernel,
        out_type=jax.ShapeDtypeStruct((NC, N), jnp.float32),
        mesh=_vector_mesh(),
        scratch_types=[
            pltpu.VMEM((NCHUNK, CH), jnp.int32),
            pltpu.VMEM((128,), jnp.float32),
            pltpu.VMEM((N,), jnp.float32),
            pltpu.VMEM_SHARED((N,), jnp.float32),
        ],
    )
    def k(dst_hbm, out_hbm, didx, ones, zbuf, acc):
        c = lax.axis_index("c")
        s = lax.axis_index("s")
        wid = c * NS + s

        one16 = jnp.ones((16,), jnp.float32)
        zero16 = jnp.zeros((16,), jnp.float32)

        @pl.loop(0, 128, step=16)
        def _(i):
            ones[pl.ds(i, 16)] = one16

        @pl.when(s == 0)
        def _():
            @pl.loop(0, N, step=16)
            def _(i):
                zbuf[pl.ds(i, 16)] = zero16
            pltpu.sync_copy(zbuf, acc)

        plsc.subcore_barrier()

        pltpu.sync_copy(dst_hbm.at[wid], didx)

        @pl.loop(0, NCHUNK)
        def _(j):
            pltpu.sync_copy(ones.at[pl.ds(0, CH)], acc.at[didx.at[j]], add=True)

        plsc.subcore_barrier()

        @pl.when(s == 0)
        def _():
            pltpu.sync_copy(acc, out_hbm.at[c])

    return k(dst3)


@jax.jit
def _sc_edge(hs, src3, dst3):
    @functools.partial(
        pl.kernel,
        out_type=jax.ShapeDtypeStruct((NC, N, H), jnp.float32),
        mesh=_vector_mesh(),
        scratch_types=[
            pltpu.VMEM((2, SCH, CH), jnp.int32),
            pltpu.VMEM((2, SCH, CH), jnp.int32),
            pltpu.VMEM((128, H), jnp.float32),
            pltpu.VMEM((128, H), jnp.float32),
            pltpu.VMEM_SHARED((NPAD, H), jnp.float32),
            pltpu.SemaphoreType.DMA,
            pltpu.SemaphoreType.DMA,
            pltpu.SemaphoreType.DMA,
            pltpu.SemaphoreType.DMA,
            pltpu.SemaphoreType.DMA,
        ],
    )
    def k(hs_hbm, src_hbm, dst_hbm, out_hbm, sidx, didx, bufa, bufb, acc,
          sga, sgb, ssa, ssb, six):
        c = lax.axis_index("c")
        s = lax.axis_index("s")
        wid = c * NS + s
        r0 = s * RT
        NG = NCHUNK // SCH

        def idx_load(g, slot):
            return (pltpu.make_async_copy(
                        src_hbm.at[wid, pl.ds(g * SCH, SCH)],
                        sidx.at[slot], six),
                    pltpu.make_async_copy(
                        dst_hbm.at[wid, pl.ds(g * SCH, SCH)],
                        didx.at[slot], six))

        pa0, pb0 = idx_load(0, 0)
        pa0.start()
        pb0.start()

        zero16 = jnp.zeros((1, 16), jnp.float32)

        @pl.loop(0, 128)
        def _(r):
            for q in range(0, H, 16):
                bufa[pl.ds(r, 1), pl.ds(q, 16)] = zero16

        @pl.loop(0, RT, step=128)
        def _(rr):
            pltpu.sync_copy(bufa, acc.at[pl.ds(r0 + rr, 128)])

        plsc.subcore_barrier()

        ba = bufa.at[pl.ds(0, CH)]
        bb = bufb.at[pl.ds(0, CH)]
        bufs = (ba, bb)
        gsem = (sga, sgb)
        ssem = (ssa, ssb)

        GH = 64

        def gstart(slot, j, bufref, sem):
            pltpu.async_copy(hs_hbm.at[sidx.at[slot, j, pl.ds(0, GH)]],
                             bufref.at[pl.ds(0, GH)], sem)
            pltpu.async_copy(hs_hbm.at[sidx.at[slot, j, pl.ds(GH, CH - GH)]],
                             bufref.at[pl.ds(GH, CH - GH)], sem)

        def gwait(slot, j, bufref, sem):
            pltpu.make_async_copy(
                hs_hbm.at[sidx.at[slot, j, pl.ds(0, GH)]],
                bufref.at[pl.ds(0, GH)], sem).wait()
            pltpu.make_async_copy(
                hs_hbm.at[sidx.at[slot, j, pl.ds(GH, CH - GH)]],
                bufref.at[pl.ds(GH, CH - GH)], sem).wait()

        rawbufs = (bufa, bufb)
        pa0.wait()
        pb0.wait()
        gstart(0, 0, bufa, sga)

        @pl.loop(0, NG, step=2)
        def _(g):
            for go in range(2):
                gc = g + go
                slot = go
                nslot = 1 - go

                for j in range(SCH):
                    b = j % 2
                    nb = 1 - b
                    if j == 0:
                        @pl.when(gc > 0)
                        def _():
                            pltpu.make_async_copy(
                                bufs[nb], acc.at[didx.at[nslot, SCH - 1]],
                                ssem[nb]).wait()
                        @pl.when(gc + 1 < NG)
                        def _():
                            fa, fb = idx_load(gc + 1, nslot)
                            fa.start()
                            fb.start()
                    else:
                        pltpu.make_async_copy(
                            bufs[nb], acc.at[didx.at[slot, j - 1]],
                            ssem[nb]).wait()
                    if j < SCH - 1:
                        gstart(slot, j + 1, rawbufs[nb], gsem[nb])
                    else:
                        @pl.when(gc + 1 < NG)
                        def _():
                            wa, wb = idx_load(gc + 1, nslot)
                            wa.wait()
                            wb.wait()
                            gstart(nslot, 0, rawbufs[nb], gsem[nb])
                    gwait(slot, j, rawbufs[b], gsem[b])
                    pltpu.async_copy(bufs[b], acc.at[didx.at[slot, j]],
                                     ssem[b], add=True)

        pltpu.make_async_copy(bufs[1], acc.at[didx.at[1, SCH - 1]],
                              ssem[1]).wait()

        plsc.subcore_barrier()

        @pl.when(s < NS - 1)
        def _():
            pltpu.sync_copy(acc.at[pl.ds(r0, RT)],
                            out_hbm.at[c, pl.ds(r0, RT)])

        @pl.when(s == NS - 1)
        def _():
            last = N - (NS - 1) * RT
            pltpu.sync_copy(acc.at[pl.ds((NS - 1) * RT, last)],
                            out_hbm.at[c, pl.ds((NS - 1) * RT, last)])

    return k(hs, src3, dst3)


BM = 2000


@jax.jit
def _tc_first(x, w, d0, d1):
    def body(x_ref, w_ref, d0_ref, d1_ref, hs_ref, dinv_ref):
        dinv = lax.rsqrt(d0_ref[...] + d1_ref[...] + 1.0)
        h = jnp.dot(x_ref[...], w_ref[...],
                    preferred_element_type=jnp.float32)
        hs_ref[...] = h * dinv
        dinv_ref[...] = dinv

    return pl.pallas_call(
        body,
        grid=(N // BM,),
        in_specs=[
            pl.BlockSpec((BM, D), lambda i: (i, 0)),
            pl.BlockSpec((D, H), lambda i: (0, 0)),
            pl.BlockSpec((BM, 1), lambda i: (i, 0)),
            pl.BlockSpec((BM, 1), lambda i: (i, 0)),
        ],
        out_specs=[
            pl.BlockSpec((BM, H), lambda i: (i, 0)),
            pl.BlockSpec((BM, 1), lambda i: (i, 0)),
        ],
        out_shape=[
            jax.ShapeDtypeStruct((N, H), jnp.float32),
            jax.ShapeDtypeStruct((N, 1), jnp.float32),
        ],
    )(x, w, d0, d1)


@jax.jit
def _tc_fuse(p0, p1, hs_prev, dinv, b, w):
    def body(p0_ref, p1_ref, hs_ref, dinv_ref, b_ref, w_ref, o_ref):
        dinv = dinv_ref[...]
        act = dinv * (p0_ref[...] + p1_ref[...] + hs_ref[...]) + b_ref[...]
        act = jnp.maximum(act, 0.0)
        o_ref[...] = jnp.dot(act, w_ref[...],
                             preferred_element_type=jnp.float32) * dinv

    return pl.pallas_call(
        body,
        grid=(N // BM,),
        in_specs=[
            pl.BlockSpec((BM, H), lambda i: (i, 0)),
            pl.BlockSpec((BM, H), lambda i: (i, 0)),
            pl.BlockSpec((BM, H), lambda i: (i, 0)),
            pl.BlockSpec((BM, 1), lambda i: (i, 0)),
            pl.BlockSpec((1, H), lambda i: (0, 0)),
            pl.BlockSpec((H, H), lambda i: (0, 0)),
        ],
        out_specs=pl.BlockSpec((BM, H), lambda i: (i, 0)),
        out_shape=jax.ShapeDtypeStruct((N, H), jnp.float32),
    )(p0, p1, hs_prev, dinv, b, w)


@jax.jit
def _tc_final(p0, p1, hs3, dinv, b3, batch2, wl, bl):
    nblk = N // BM

    def body(p0_ref, p1_ref, hs_ref, dinv_ref, b_ref, batch_ref, wl_ref,
             bl_ref, o_ref, psum, csum):
        i = pl.program_id(0)

        @pl.when(i == 0)
        def _():
            psum[...] = jnp.zeros_like(psum)
            csum[...] = jnp.zeros_like(csum)

        h3 = dinv_ref[...] * (p0_ref[...] + p1_ref[...] + hs_ref[...]) \
            + b_ref[...]
        seg = lax.broadcasted_iota(jnp.int32, (BM, G), 1)
        m = (batch_ref[...] == seg).astype(jnp.float32)
        dn = (((0,), (0,)), ((), ()))
        psum[...] += lax.dot_general(m, h3, dn,
                                     preferred_element_type=jnp.float32)
        csum[...] += lax.dot_general(m, jnp.ones((BM, 1), jnp.float32), dn,
                                     preferred_element_type=jnp.float32)

        @pl.when(i == nblk - 1)
        def _():
            pooled = psum[...] / jnp.maximum(csum[...], 1.0)
            o_ref[...] = jnp.dot(pooled, wl_ref[...],
                                 preferred_element_type=jnp.float32) \
                + bl_ref[...]

    return pl.pallas_call(
        body,
        grid=(nblk,),
        in_specs=[
            pl.BlockSpec((BM, H), lambda i: (i, 0)),
            pl.BlockSpec((BM, H), lambda i: (i, 0)),
            pl.BlockSpec((BM, H), lambda i: (i, 0)),
            pl.BlockSpec((BM, 1), lambda i: (i, 0)),
            pl.BlockSpec((1, H), lambda i: (0, 0)),
            pl.BlockSpec((BM, 1), lambda i: (i, 0)),
            pl.BlockSpec((H, C), lambda i: (0, 0)),
            pl.BlockSpec((1, C), lambda i: (0, 0)),
        ],
        out_specs=pl.BlockSpec((G, C), lambda i: (0, 0)),
        out_shape=jax.ShapeDtypeStruct((G, C), jnp.float32),
        scratch_shapes=[
            pltpu.VMEM((G, H), jnp.float32),
            pltpu.VMEM((G, 1), jnp.float32),
        ],
    )(p0, p1, hs3, dinv, b3, batch2, wl, bl)


def kernel(x, edge_index, batch, W1, b1, W2, b2, W3, b3, Wl, bl):
    src3 = edge_index[0].reshape(NW, NCHUNK, CH)
    dst3 = edge_index[1].reshape(NW, NCHUNK, CH)

    degp = _sc_degree(dst3)
    d0 = degp[0].reshape(N, 1)
    d1 = degp[1].reshape(N, 1)
    hs1, dinv = _tc_first(x, W1, d0, d1)

    p = _sc_edge(hs1, src3, dst3)
    hs2 = _tc_fuse(p[0], p[1], hs1, dinv, b1.reshape(1, H), W2)
    p = _sc_edge(hs2, src3, dst3)
    hs3 = _tc_fuse(p[0], p[1], hs2, dinv, b2.reshape(1, H), W3)
    p = _sc_edge(hs3, src3, dst3)

    return _tc_final(p[0], p[1], hs3, dinv, b3.reshape(1, H),
                     batch.reshape(N, 1), Wl, bl.reshape(1, C))


# speedup vs baseline: 1.0066x; 1.0066x over previous
import functools

import jax
import jax.numpy as jnp
from jax import lax
from jax.experimental import pallas as pl
from jax.experimental.pallas import tpu as pltpu
from jax.experimental.pallas import tpu_sc as plsc

N = 10000
E = 320000
D = 128
H = 128
C = 10
G = 64

NC = 2
NS = 16
NW = NC * NS
EPW = E // NW
CH = 125
NCHUNK = EPW // CH
SCH = 8
NPAD = 10240
RT = NPAD // NS

@functools.lru_cache(maxsize=1)
def _vector_mesh():
    return plsc.VectorSubcoreMesh(
        core_axis_name="c", subcore_axis_name="s",
        num_cores=NC, num_subcores=NS)


@jax.jit
def _sc_degree(dst3):
    @functools.partial(
        pl.kernel,
        out_type=jax.ShapeDtypeStruct((NC, N), jnp.float32),
        mesh=_vector_mesh(),
        scratch_types=[
            pltpu.VMEM((NCHUNK, CH), jnp.int32),
            pltpu.VMEM((128,), jnp.float32),
            pltpu.VMEM((N,), jnp.float32),
            pltpu.VMEM_SHARED((N,), jnp.float32),
        ],
    )
    def k(dst_hbm, out_hbm, didx, ones, zbuf, acc):
        c = lax.axis_index("c")
        s = lax.axis_index("s")
        wid = c * NS + s

        one16 = jnp.ones((16,), jnp.float32)
        zero16 = jnp.zeros((16,), jnp.float32)

        @pl.loop(0, 128, step=16)
        def _(i):
            ones[pl.ds(i, 16)] = one16

        @pl.when(s == 0)
        def _():
            @pl.loop(0, N, step=16)
            def _(i):
                zbuf[pl.ds(i, 16)] = zero16
            pltpu.sync_copy(zbuf, acc)

        plsc.subcore_barrier()

        pltpu.sync_copy(dst_hbm.at[wid], didx)

        @pl.loop(0, NCHUNK)
        def _(j):
            pltpu.sync_copy(ones.at[pl.ds(0, CH)], acc.at[didx.at[j]], add=True)

        plsc.subcore_barrier()

        @pl.when(s == 0)
        def _():
            pltpu.sync_copy(acc, out_hbm.at[c])

    return k(dst3)


@jax.jit
def _sc_edge(hs, src3, dst3):
    @functools.partial(
        pl.kernel,
        out_type=jax.ShapeDtypeStruct((NC, N, H), jnp.float32),
        mesh=_vector_mesh(),
        scratch_types=[
            pltpu.VMEM((2, SCH, CH), jnp.int32),
            pltpu.VMEM((2, SCH, CH), jnp.int32),
            pltpu.VMEM((128, H), jnp.float32),
            pltpu.VMEM((128, H), jnp.float32),
            pltpu.VMEM_SHARED((NPAD, H), jnp.float32),
            pltpu.SemaphoreType.DMA,
            pltpu.SemaphoreType.DMA,
            pltpu.SemaphoreType.DMA,
            pltpu.SemaphoreType.DMA,
            pltpu.SemaphoreType.DMA,
        ],
    )
    def k(hs_hbm, src_hbm, dst_hbm, out_hbm, sidx, didx, bufa, bufb, acc,
          sga, sgb, ssa, ssb, six):
        c = lax.axis_index("c")
        s = lax.axis_index("s")
        wid = c * NS + s
        r0 = s * RT
        NG = NCHUNK // SCH

        def idx_load(g, slot):
            return (pltpu.make_async_copy(
                        src_hbm.at[wid, pl.ds(g * SCH, SCH)],
                        sidx.at[slot], six),
                    pltpu.make_async_copy(
                        dst_hbm.at[wid, pl.ds(g * SCH, SCH)],
                        didx.at[slot], six))

        pa0, pb0 = idx_load(0, 0)
        pa0.start()
        pb0.start()

        ba = bufa.at[pl.ds(0, CH)]
        bb = bufb.at[pl.ds(0, CH)]
        bufs = (ba, bb)
        gsem = (sga, sgb)
        ssem = (ssa, ssb)

        GH = 64

        def gstart(slot, j, bufref, sem):
            pltpu.async_copy(hs_hbm.at[sidx.at[slot, j, pl.ds(0, GH)]],
                             bufref.at[pl.ds(0, GH)], sem)
            pltpu.async_copy(hs_hbm.at[sidx.at[slot, j, pl.ds(GH, CH - GH)]],
                             bufref.at[pl.ds(GH, CH - GH)], sem)

        def gwait(slot, j, bufref, sem):
            pltpu.make_async_copy(
                hs_hbm.at[sidx.at[slot, j, pl.ds(0, GH)]],
                bufref.at[pl.ds(0, GH)], sem).wait()
            pltpu.make_async_copy(
                hs_hbm.at[sidx.at[slot, j, pl.ds(GH, CH - GH)]],
                bufref.at[pl.ds(GH, CH - GH)], sem).wait()

        rawbufs = (bufa, bufb)

        pa0.wait()
        pb0.wait()
        gstart(0, 0, bufb, sgb)

        zero16 = jnp.zeros((1, 16), jnp.float32)

        @pl.loop(0, 128)
        def _(r):
            for q in range(0, H, 16):
                bufa[pl.ds(r, 1), pl.ds(q, 16)] = zero16

        @pl.loop(0, RT, step=128)
        def _(rr):
            pltpu.sync_copy(bufa, acc.at[pl.ds(r0 + rr, 128)])

        gstart(0, 1, bufa, sga)

        plsc.subcore_barrier()

        @pl.loop(0, NG, step=2)
        def _(g):
            for go in range(2):
                gc = g + go
                slot = go
                nslot = 1 - go

                for j in range(SCH):
                    b = (j + 1) % 2
                    nb = 1 - b
                    if j == 0:
                        @pl.when(gc > 0)
                        def _():
                            pltpu.make_async_copy(
                                bufs[nb], acc.at[didx.at[nslot, SCH - 1]],
                                ssem[nb]).wait()
                        @pl.when(gc + 1 < NG)
                        def _():
                            fa, fb = idx_load(gc + 1, nslot)
                            fa.start()
                            fb.start()
                    else:
                        pltpu.make_async_copy(
                            bufs[nb], acc.at[didx.at[slot, j - 1]],
                            ssem[nb]).wait()
                    if j == 0:
                        @pl.when(gc > 0)
                        def _():
                            gstart(slot, 1, rawbufs[nb], gsem[nb])
                    elif j < SCH - 1:
                        gstart(slot, j + 1, rawbufs[nb], gsem[nb])
                    else:
                        @pl.when(gc + 1 < NG)
                        def _():
                            wa, wb = idx_load(gc + 1, nslot)
                            wa.wait()
                            wb.wait()
                            gstart(nslot, 0, rawbufs[nb], gsem[nb])
                    gwait(slot, j, rawbufs[b], gsem[b])
                    pltpu.async_copy(bufs[b], acc.at[didx.at[slot, j]],
                                     ssem[b], add=True)

        pltpu.make_async_copy(bufs[0], acc.at[didx.at[1, SCH - 1]],
                              ssem[0]).wait()

        plsc.subcore_barrier()

        @pl.when(s < NS - 1)
        def _():
            pltpu.sync_copy(acc.at[pl.ds(r0, RT)],
                            out_hbm.at[c, pl.ds(r0, RT)])

        @pl.when(s == NS - 1)
        def _():
            last = N - (NS - 1) * RT
            pltpu.sync_copy(acc.at[pl.ds((NS - 1) * RT, last)],
                            out_hbm.at[c, pl.ds((NS - 1) * RT, last)])

    return k(hs, src3, dst3)


BM = 2000


@jax.jit
def _tc_mm(x, w):
    def body(x_ref, w_ref, o_ref):
        o_ref[...] = jnp.dot(x_ref[...], w_ref[...],
                             preferred_element_type=jnp.float32)

    return pl.pallas_call(
        body,
        grid=(N // BM,),
        in_specs=[
            pl.BlockSpec((BM, D), lambda i: (i, 0)),
            pl.BlockSpec((D, H), lambda i: (0, 0)),
        ],
        out_specs=pl.BlockSpec((BM, H), lambda i: (i, 0)),
        out_shape=jax.ShapeDtypeStruct((N, H), jnp.float32),
    )(x, w)


@jax.jit
def _tc_scale(h1, d0, d1):
    def body(h_ref, d0_ref, d1_ref, hs_ref, dinv_ref):
        dinv = lax.rsqrt(d0_ref[...] + d1_ref[...] + 1.0)
        hs_ref[...] = h_ref[...] * dinv
        dinv_ref[...] = dinv

    return pl.pallas_call(
        body,
        grid=(N // BM,),
        in_specs=[
            pl.BlockSpec((BM, H), lambda i: (i, 0)),
            pl.BlockSpec((BM, 1), lambda i: (i, 0)),
            pl.BlockSpec((BM, 1), lambda i: (i, 0)),
        ],
        out_specs=[
            pl.BlockSpec((BM, H), lambda i: (i, 0)),
            pl.BlockSpec((BM, 1), lambda i: (i, 0)),
        ],
        out_shape=[
            jax.ShapeDtypeStruct((N, H), jnp.float32),
            jax.ShapeDtypeStruct((N, 1), jnp.float32),
        ],
    )(h1, d0, d1)


@jax.jit
def _tc_fuse(p0, p1, hs_prev, dinv, b, w):
    def body(p0_ref, p1_ref, hs_ref, dinv_ref, b_ref, w_ref, o_ref):
        dinv = dinv_ref[...]
        act = dinv * (p0_ref[...] + p1_ref[...] + hs_ref[...]) + b_ref[...]
        act = jnp.maximum(act, 0.0)
        o_ref[...] = jnp.dot(act, w_ref[...],
                             preferred_element_type=jnp.float32) * dinv

    return pl.pallas_call(
        body,
        grid=(N // BM,),
        in_specs=[
            pl.BlockSpec((BM, H), lambda i: (i, 0)),
            pl.BlockSpec((BM, H), lambda i: (i, 0)),
            pl.BlockSpec((BM, H), lambda i: (i, 0)),
            pl.BlockSpec((BM, 1), lambda i: (i, 0)),
            pl.BlockSpec((1, H), lambda i: (0, 0)),
            pl.BlockSpec((H, H), lambda i: (0, 0)),
        ],
        out_specs=pl.BlockSpec((BM, H), lambda i: (i, 0)),
        out_shape=jax.ShapeDtypeStruct((N, H), jnp.float32),
    )(p0, p1, hs_prev, dinv, b, w)


@jax.jit
def _tc_final(p0, p1, hs3, dinv, b3, batch2, wl, bl):
    nblk = N // BM

    def body(p0_ref, p1_ref, hs_ref, dinv_ref, b_ref, batch_ref, wl_ref,
             bl_ref, o_ref, psum, csum):
        i = pl.program_id(0)

        @pl.when(i == 0)
        def _():
            psum[...] = jnp.zeros_like(psum)
            csum[...] = jnp.zeros_like(csum)

        h3 = dinv_ref[...] * (p0_ref[...] + p1_ref[...] + hs_ref[...]) \
            + b_ref[...]
        seg = lax.broadcasted_iota(jnp.int32, (BM, G), 1)
        m = (batch_ref[...] == seg).astype(jnp.float32)
        dn = (((0,), (0,)), ((), ()))
        psum[...] += lax.dot_general(m, h3, dn,
                                     preferred_element_type=jnp.float32)
        csum[...] += lax.dot_general(m, jnp.ones((BM, 1), jnp.float32), dn,
                                     preferred_element_type=jnp.float32)

        @pl.when(i == nblk - 1)
        def _():
            pooled = psum[...] / jnp.maximum(csum[...], 1.0)
            o_ref[...] = jnp.dot(pooled, wl_ref[...],
                                 preferred_element_type=jnp.float32) \
                + bl_ref[...]

    return pl.pallas_call(
        body,
        grid=(nblk,),
        in_specs=[
            pl.BlockSpec((BM, H), lambda i: (i, 0)),
            pl.BlockSpec((BM, H), lambda i: (i, 0)),
            pl.BlockSpec((BM, H), lambda i: (i, 0)),
            pl.BlockSpec((BM, 1), lambda i: (i, 0)),
            pl.BlockSpec((1, H), lambda i: (0, 0)),
            pl.BlockSpec((BM, 1), lambda i: (i, 0)),
            pl.BlockSpec((H, C), lambda i: (0, 0)),
            pl.BlockSpec((1, C), lambda i: (0, 0)),
        ],
        out_specs=pl.BlockSpec((G, C), lambda i: (0, 0)),
        out_shape=jax.ShapeDtypeStruct((G, C), jnp.float32),
        scratch_shapes=[
            pltpu.VMEM((G, H), jnp.float32),
            pltpu.VMEM((G, 1), jnp.float32),
        ],
    )(p0, p1, hs3, dinv, b3, batch2, wl, bl)


def kernel(x, edge_index, batch, W1, b1, W2, b2, W3, b3, Wl, bl):
    src3 = edge_index[0].reshape(NW, NCHUNK, CH)
    dst3 = edge_index[1].reshape(NW, NCHUNK, CH)

    degp = _sc_degree(dst3)
    h1 = _tc_mm(x, W1)
    d0 = degp[0].reshape(N, 1)
    d1 = degp[1].reshape(N, 1)
    hs1, dinv = _tc_scale(h1, d0, d1)

    p = _sc_edge(hs1, src3, dst3)
    hs2 = _tc_fuse(p[0], p[1], hs1, dinv, b1.reshape(1, H), W2)
    p = _sc_edge(hs2, src3, dst3)
    hs3 = _tc_fuse(p[0], p[1], hs2, dinv, b2.reshape(1, H), W3)
    p = _sc_edge(hs3, src3, dst3)

    return _tc_final(p[0], p[1], hs3, dinv, b3.reshape(1, H),
                     batch.reshape(N, 1), Wl, bl.reshape(1, C))


# speedup vs baseline: 1.0840x; 1.0769x over previous
import functools

import jax
import jax.numpy as jnp
from jax import lax
from jax.experimental import pallas as pl
from jax.experimental.pallas import tpu as pltpu
from jax.experimental.pallas import tpu_sc as plsc

N = 10000
E = 320000
D = 128
H = 128
C = 10
G = 64

NC = 2
NS = 16
NW = NC * NS
EPW = E // NW
CH = 125
NCHUNK = EPW // CH
SCH = 8
NPAD = 10240
RT = NPAD // NS

@functools.lru_cache(maxsize=1)
def _vector_mesh():
    return plsc.VectorSubcoreMesh(
        core_axis_name="c", subcore_axis_name="s",
        num_cores=NC, num_subcores=NS)


@jax.jit
def _sc_degree(e3):
    @functools.partial(
        pl.kernel,
        out_type=jax.ShapeDtypeStruct((NC, N), jnp.float32),
        mesh=_vector_mesh(),
        scratch_types=[
            pltpu.VMEM((NCHUNK, CH), jnp.int32),
            pltpu.VMEM((128,), jnp.float32),
            pltpu.VMEM((N,), jnp.float32),
            pltpu.VMEM_SHARED((N,), jnp.float32),
        ],
    )
    def k(e3_hbm, out_hbm, didx, ones, zbuf, acc):
        c = lax.axis_index("c")
        s = lax.axis_index("s")
        wid = c * NS + s

        one16 = jnp.ones((16,), jnp.float32)
        zero16 = jnp.zeros((16,), jnp.float32)

        @pl.loop(0, 128, step=16)
        def _(i):
            ones[pl.ds(i, 16)] = one16

        @pl.when(s == 0)
        def _():
            @pl.loop(0, N, step=16)
            def _(i):
                zbuf[pl.ds(i, 16)] = zero16
            pltpu.sync_copy(zbuf, acc)

        plsc.subcore_barrier()

        pltpu.sync_copy(e3_hbm.at[1, wid], didx)

        @pl.loop(0, NCHUNK)
        def _(j):
            pltpu.sync_copy(ones.at[pl.ds(0, CH)], acc.at[didx.at[j]], add=True)

        plsc.subcore_barrier()

        @pl.when(s == 0)
        def _():
            pltpu.sync_copy(acc, out_hbm.at[c])

    return k(e3)


@jax.jit
def _sc_edge(hs, e3):
    @functools.partial(
        pl.kernel,
        out_type=jax.ShapeDtypeStruct((NC, N, H), jnp.float32),
        mesh=_vector_mesh(),
        scratch_types=[
            pltpu.VMEM((2, SCH, CH), jnp.int32),
            pltpu.VMEM((2, SCH, CH), jnp.int32),
            pltpu.VMEM((128, H), jnp.float32),
            pltpu.VMEM((128, H), jnp.float32),
            pltpu.VMEM_SHARED((NPAD, H), jnp.float32),
            pltpu.SemaphoreType.DMA,
            pltpu.SemaphoreType.DMA,
            pltpu.SemaphoreType.DMA,
            pltpu.SemaphoreType.DMA,
            pltpu.SemaphoreType.DMA,
        ],
    )
    def k(hs_hbm, e3_hbm, out_hbm, sidx, didx, bufa, bufb, acc,
          sga, sgb, ssa, ssb, six):
        c = lax.axis_index("c")
        s = lax.axis_index("s")
        wid = c * NS + s
        r0 = s * RT
        NG = NCHUNK // SCH

        def idx_load(g, slot):
            return (pltpu.make_async_copy(
                        e3_hbm.at[0, wid, pl.ds(g * SCH, SCH)],
                        sidx.at[slot], six),
                    pltpu.make_async_copy(
                        e3_hbm.at[1, wid, pl.ds(g * SCH, SCH)],
                        didx.at[slot], six))

        pa0, pb0 = idx_load(0, 0)
        pa0.start()
        pb0.start()

        ba = bufa.at[pl.ds(0, CH)]
        bb = bufb.at[pl.ds(0, CH)]
        bufs = (ba, bb)
        gsem = (sga, sgb)
        ssem = (ssa, ssb)

        GH = 64

        def gstart(slot, j, bufref, sem):
            pltpu.async_copy(hs_hbm.at[sidx.at[slot, j, pl.ds(0, GH)]],
                             bufref.at[pl.ds(0, GH)], sem)
            pltpu.async_copy(hs_hbm.at[sidx.at[slot, j, pl.ds(GH, CH - GH)]],
                             bufref.at[pl.ds(GH, CH - GH)], sem)

        def gwait(slot, j, bufref, sem):
            pltpu.make_async_copy(
                hs_hbm.at[sidx.at[slot, j, pl.ds(0, GH)]],
                bufref.at[pl.ds(0, GH)], sem).wait()
            pltpu.make_async_copy(
                hs_hbm.at[sidx.at[slot, j, pl.ds(GH, CH - GH)]],
                bufref.at[pl.ds(GH, CH - GH)], sem).wait()

        rawbufs = (bufa, bufb)

        pa0.wait()
        pb0.wait()
        gstart(0, 0, bufb, sgb)

        zero16 = jnp.zeros((1, 16), jnp.float32)

        @pl.loop(0, 128)
        def _(r):
            for q in range(0, H, 16):
                bufa[pl.ds(r, 1), pl.ds(q, 16)] = zero16

        @pl.loop(0, RT, step=128)
        def _(rr):
            pltpu.sync_copy(bufa, acc.at[pl.ds(r0 + rr, 128)])

        gstart(0, 1, bufa, sga)

        plsc.subcore_barrier()

        @pl.loop(0, NG, step=2)
        def _(g):
            for go in range(2):
                gc = g + go
                slot = go
                nslot = 1 - go

                for j in range(SCH):
                    b = (j + 1) % 2
                    nb = 1 - b
                    if j == 0:
                        @pl.when(gc > 0)
                        def _():
                            pltpu.make_async_copy(
                                bufs[nb], acc.at[didx.at[nslot, SCH - 1]],
                                ssem[nb]).wait()
                        @pl.when(gc + 1 < NG)
                        def _():
                            fa, fb = idx_load(gc + 1, nslot)
                            fa.start()
                            fb.start()
                    else:
                        pltpu.make_async_copy(
                            bufs[nb], acc.at[didx.at[slot, j - 1]],
                            ssem[nb]).wait()
                    if j == 0:
                        @pl.when(gc > 0)
                        def _():
                            gstart(slot, 1, rawbufs[nb], gsem[nb])
                    elif j < SCH - 1:
                        gstart(slot, j + 1, rawbufs[nb], gsem[nb])
                    else:
                        @pl.when(gc + 1 < NG)
                        def _():
                            wa, wb = idx_load(gc + 1, nslot)
                            wa.wait()
                            wb.wait()
                            gstart(nslot, 0, rawbufs[nb], gsem[nb])
                    gwait(slot, j, rawbufs[b], gsem[b])
                    pltpu.async_copy(bufs[b], acc.at[didx.at[slot, j]],
                                     ssem[b], add=True)

        pltpu.make_async_copy(bufs[0], acc.at[didx.at[1, SCH - 1]],
                              ssem[0]).wait()

        plsc.subcore_barrier()

        @pl.when(s < NS - 1)
        def _():
            pltpu.sync_copy(acc.at[pl.ds(r0, RT)],
                            out_hbm.at[c, pl.ds(r0, RT)])

        @pl.when(s == NS - 1)
        def _():
            last = N - (NS - 1) * RT
            pltpu.sync_copy(acc.at[pl.ds((NS - 1) * RT, last)],
                            out_hbm.at[c, pl.ds((NS - 1) * RT, last)])

    return k(hs, e3)


BM = 2000


@jax.jit
def _tc_mm(x, w):
    def body(x_ref, w_ref, o_ref):
        o_ref[...] = jnp.dot(x_ref[...], w_ref[...],
                             preferred_element_type=jnp.float32)

    return pl.pallas_call(
        body,
        grid=(N // BM,),
        in_specs=[
            pl.BlockSpec((BM, D), lambda i: (i, 0)),
            pl.BlockSpec((D, H), lambda i: (0, 0)),
        ],
        out_specs=pl.BlockSpec((BM, H), lambda i: (i, 0)),
        out_shape=jax.ShapeDtypeStruct((N, H), jnp.float32),
    )(x, w)


@jax.jit
def _tc_scale(h1, d0, d1):
    def body(h_ref, d0_ref, d1_ref, hs_ref, dinv_ref):
        dinv = lax.rsqrt(d0_ref[...] + d1_ref[...] + 1.0)
        hs_ref[...] = h_ref[...] * dinv
        dinv_ref[...] = dinv

    return pl.pallas_call(
        body,
        grid=(N // BM,),
        in_specs=[
            pl.BlockSpec((BM, H), lambda i: (i, 0)),
            pl.BlockSpec((BM, 1), lambda i: (i, 0)),
            pl.BlockSpec((BM, 1), lambda i: (i, 0)),
        ],
        out_specs=[
            pl.BlockSpec((BM, H), lambda i: (i, 0)),
            pl.BlockSpec((BM, 1), lambda i: (i, 0)),
        ],
        out_shape=[
            jax.ShapeDtypeStruct((N, H), jnp.float32),
            jax.ShapeDtypeStruct((N, 1), jnp.float32),
        ],
    )(h1, d0, d1)


@jax.jit
def _tc_fuse(parts, hs_prev, dinv, b, w):
    def body(p0_ref, p1_ref, hs_ref, dinv_ref, b_ref, w_ref, o_ref):
        dinv = dinv_ref[...]
        act = dinv * (p0_ref[0] + p1_ref[0] + hs_ref[...]) + b_ref[...]
        act = jnp.maximum(act, 0.0)
        o_ref[...] = jnp.dot(act, w_ref[...],
                             preferred_element_type=jnp.float32) * dinv

    return pl.pallas_call(
        body,
        grid=(N // BM,),
        in_specs=[
            pl.BlockSpec((1, BM, H), lambda i: (0, i, 0)),
            pl.BlockSpec((1, BM, H), lambda i: (1, i, 0)),
            pl.BlockSpec((BM, H), lambda i: (i, 0)),
            pl.BlockSpec((BM, 1), lambda i: (i, 0)),
            pl.BlockSpec((1, H), lambda i: (0, 0)),
            pl.BlockSpec((H, H), lambda i: (0, 0)),
        ],
        out_specs=pl.BlockSpec((BM, H), lambda i: (i, 0)),
        out_shape=jax.ShapeDtypeStruct((N, H), jnp.float32),
    )(parts, parts, hs_prev, dinv, b, w)


@jax.jit
def _tc_final(parts, hs3, dinv, b3, batch_row, wl, bl):
    nblk = N // BM

    def body(p0_ref, p1_ref, hs_ref, dinv_ref, b_ref, batch_ref, wl_ref,
             bl_ref, o_ref, psum, csum):
        i = pl.program_id(0)

        @pl.when(i == 0)
        def _():
            psum[...] = jnp.zeros_like(psum)
            csum[...] = jnp.zeros_like(csum)

        h3 = dinv_ref[...] * (p0_ref[0] + p1_ref[0] + hs_ref[...]) \
            + b_ref[...]
        seg = lax.broadcasted_iota(jnp.int32, (G, BM), 0)
        m = (batch_ref[0] == seg).astype(jnp.float32)
        dn = (((1,), (0,)), ((), ()))
        psum[...] += lax.dot_general(m, h3, dn,
                                     preferred_element_type=jnp.float32)
        csum[...] += lax.dot_general(m, jnp.ones((BM, 1), jnp.float32), dn,
                                     preferred_element_type=jnp.float32)

        @pl.when(i == nblk - 1)
        def _():
            pooled = psum[...] / jnp.maximum(csum[...], 1.0)
            o_ref[...] = jnp.dot(pooled, wl_ref[...],
                                 preferred_element_type=jnp.float32) \
                + bl_ref[...]

    return pl.pallas_call(
        body,
        grid=(nblk,),
        in_specs=[
            pl.BlockSpec((1, BM, H), lambda i: (0, i, 0)),
            pl.BlockSpec((1, BM, H), lambda i: (1, i, 0)),
            pl.BlockSpec((BM, H), lambda i: (i, 0)),
            pl.BlockSpec((BM, 1), lambda i: (i, 0)),
            pl.BlockSpec((1, H), lambda i: (0, 0)),
            pl.BlockSpec((1, 1, BM), lambda i: (i, 0, 0)),
            pl.BlockSpec((H, C), lambda i: (0, 0)),
            pl.BlockSpec((1, C), lambda i: (0, 0)),
        ],
        out_specs=pl.BlockSpec((G, C), lambda i: (0, 0)),
        out_shape=jax.ShapeDtypeStruct((G, C), jnp.float32),
        scratch_shapes=[
            pltpu.VMEM((G, H), jnp.float32),
            pltpu.VMEM((G, 1), jnp.float32),
        ],
    )(parts, parts, hs3, dinv, b3, batch_row, wl, bl)


def kernel(x, edge_index, batch, W1, b1, W2, b2, W3, b3, Wl, bl):
    e3 = edge_index.reshape(2, NW, NCHUNK, CH)

    degp = _sc_degree(e3)
    h1 = _tc_mm(x, W1)
    d0 = degp[0].reshape(N, 1)
    d1 = degp[1].reshape(N, 1)
    hs1, dinv = _tc_scale(h1, d0, d1)

    p = _sc_edge(hs1, e3)
    hs2 = _tc_fuse(p, hs1, dinv, b1.reshape(1, H), W2)
    p = _sc_edge(hs2, e3)
    hs3 = _tc_fuse(p, hs2, dinv, b2.reshape(1, H), W3)
    p = _sc_edge(hs3, e3)

    return _tc_final(p, hs3, dinv, b3.reshape(1, H),
                     batch.reshape(N // BM, 1, BM), Wl, bl.reshape(1, C))
